# baseline (speedup 1.0000x reference)
import jax
import jax.numpy as jnp
from jax.experimental import pallas as pl
from jax.experimental.pallas import tpu as pltpu

_CHUNKS = 4


def _copy_body(x_ref, e_ref, x_out_ref, e_out_ref, sem_x, sem_e):
    nx = x_ref.shape[0] // _CHUNKS
    ne = e_ref.shape[1] // _CHUNKS
    copies = []
    for c in range(_CHUNKS):
        cx = pltpu.make_async_copy(
            x_ref.at[pl.ds(c * nx, nx), :], x_out_ref.at[pl.ds(c * nx, nx), :],
            sem_x.at[c])
        ce = pltpu.make_async_copy(
            e_ref.at[:, pl.ds(c * ne, ne)], e_out_ref.at[:, pl.ds(c * ne, ne)],
            sem_e.at[c])
        cx.start()
        ce.start()
        copies.append((cx, ce))
    for cx, ce in copies:
        cx.wait()
        ce.wait()


def kernel(x, edge_index, edge_attr):
    del edge_index
    e_t = edge_attr.T
    x_out, e_out_t = pl.pallas_call(
        _copy_body,
        out_shape=(
            jax.ShapeDtypeStruct(x.shape, x.dtype),
            jax.ShapeDtypeStruct(e_t.shape, e_t.dtype),
        ),
        in_specs=[
            pl.BlockSpec(memory_space=pl.ANY),
            pl.BlockSpec(memory_space=pl.ANY),
        ],
        out_specs=(
            pl.BlockSpec(memory_space=pl.ANY),
            pl.BlockSpec(memory_space=pl.ANY),
        ),
        scratch_shapes=[
            pltpu.SemaphoreType.DMA((_CHUNKS,)),
            pltpu.SemaphoreType.DMA((_CHUNKS,)),
        ],
    )(x, e_t)
    return (x_out, e_out_t.T)


# speedup vs baseline: 47.1762x; 47.1762x over previous
import jax
import jax.numpy as jnp
from jax.experimental import pallas as pl


def _copy_body(x_ref, e_ref, x_out_ref, e_out_ref):
    x_out_ref[...] = x_ref[...]
    e_out_ref[...] = e_ref[...]


def kernel(x, edge_index, edge_attr):
    del edge_index
    n_nodes, d_feat = x.shape
    n_edges, d_edge = edge_attr.shape
    e_t = edge_attr.T

    grid = 2
    bx = n_nodes // grid
    be = n_edges // grid

    x_out, e_out_t = pl.pallas_call(
        _copy_body,
        grid=(grid,),
        out_shape=(
            jax.ShapeDtypeStruct(x.shape, x.dtype),
            jax.ShapeDtypeStruct(e_t.shape, e_t.dtype),
        ),
        in_specs=[
            pl.BlockSpec((bx, d_feat), lambda i: (i, 0)),
            pl.BlockSpec((d_edge, be), lambda i: (0, i)),
        ],
        out_specs=(
            pl.BlockSpec((bx, d_feat), lambda i: (i, 0)),
            pl.BlockSpec((d_edge, be), lambda i: (0, i)),
        ),
    )(x, e_t)
    return (x_out, e_out_t.T)


# speedup vs baseline: 48.5321x; 1.0287x over previous
import jax
import jax.numpy as jnp
from jax.experimental import pallas as pl
from jax.experimental.pallas import tpu as pltpu


def _copy_body(x_ref, e_ref, x_out_ref, e_out_ref):
    x_out_ref[...] = x_ref[...]
    e_out_ref[...] = e_ref[...]


def kernel(x, edge_index, edge_attr):
    del edge_index
    n_nodes, d_feat = x.shape
    n_edges, d_edge = edge_attr.shape
    e_t = edge_attr.T

    grid = 2
    bx = n_nodes // grid
    be = n_edges // grid

    x_out, e_out_t = pl.pallas_call(
        _copy_body,
        grid=(grid,),
        out_shape=(
            jax.ShapeDtypeStruct(x.shape, x.dtype),
            jax.ShapeDtypeStruct(e_t.shape, e_t.dtype),
        ),
        in_specs=[
            pl.BlockSpec((bx, d_feat), lambda i: (i, 0)),
            pl.BlockSpec((d_edge, be), lambda i: (0, i)),
        ],
        out_specs=(
            pl.BlockSpec((bx, d_feat), lambda i: (i, 0)),
            pl.BlockSpec((d_edge, be), lambda i: (0, i)),
        ),
        compiler_params=pltpu.CompilerParams(
            dimension_semantics=("parallel",),
        ),
    )(x, e_t)
    return (x_out, e_out_t.T)
